# baseline (speedup 1.0000x reference)
import jax
import jax.numpy as jnp
from jax import lax
from jax.experimental import pallas as pl
from jax.experimental.pallas import tpu as pltpu
from jax.experimental.pallas import tpu_sc as plsc

_NC, _NS = 2, 16
_NW = _NC * _NS
_CHUNK = 128
_SEL_ROWS = 1024


def _sc_gather_wide(w_wide, idx, N, D):
    C = _CHUNK
    b_per_w = N // _NW
    mesh = plsc.VectorSubcoreMesh(core_axis_name="c", subcore_axis_name="s")

    @pl.kernel(
        out_type=jax.ShapeDtypeStruct((N, 2 * D), jnp.float32),
        mesh=mesh,
        scratch_types=[
            pltpu.VMEM((C,), jnp.int32),
            pltpu.VMEM((C, 2 * D), jnp.float32),
            pltpu.SemaphoreType.DMA,
        ],
    )
    def _gather(w_hbm, i_hbm, o_hbm, i_v, s_v, sem):
        wid = lax.axis_index("s") * _NC + lax.axis_index("c")
        base = wid * b_per_w

        @pl.loop(0, b_per_w, step=C)
        def _(off):
            pltpu.sync_copy(i_hbm.at[pl.ds(base + off, C)], i_v)

            @pl.loop(0, C, step=16)
            def _(k):
                i_v[pl.ds(k, 16)] = lax.shift_right_logical(
                    i_v[pl.ds(k, 16)], 1)

            pltpu.async_copy(w_hbm.at[plsc.Indices(i_v)], s_v, sem).wait()
            pltpu.sync_copy(s_v, o_hbm.at[pl.ds(base + off, C)])

    return _gather(w_wide, idx)


def _tc_select_half(wide, parity, N, D):
    R = _SEL_ROWS

    def body(w_ref, p_ref, o_ref):
        lo = w_ref[:, :D]
        hi = w_ref[:, D:]
        o_ref[...] = jnp.where(p_ref[...] != 0, hi, lo)

    return pl.pallas_call(
        body,
        grid=(N // R,),
        in_specs=[
            pl.BlockSpec((R, 2 * D), lambda i: (i, 0)),
            pl.BlockSpec((R, 1), lambda i: (i, 0)),
        ],
        out_specs=pl.BlockSpec((R, D), lambda i: (i, 0)),
        out_shape=jax.ShapeDtypeStruct((N, D), jnp.float32),
    )(wide, parity)


def kernel(token_ids, W):
    B, L = token_ids.shape
    V, D = W.shape
    N = B * L
    assert N % (_NW * _CHUNK) == 0 and V % 2 == 0 and N % _SEL_ROWS == 0

    idx = token_ids.reshape(N)
    w_wide = W.reshape(V // 2, 2 * D)
    wide = _sc_gather_wide(w_wide, idx, N, D)
    parity = lax.bitwise_and(idx, 1).reshape(N, 1)
    out = _tc_select_half(wide, parity, N, D)
    return out.reshape(B, L, D)


# speedup vs baseline: 1.0726x; 1.0726x over previous
import jax
import jax.numpy as jnp
from jax import lax
from jax.experimental import pallas as pl
from jax.experimental.pallas import tpu as pltpu
from jax.experimental.pallas import tpu_sc as plsc

_NC, _NS = 2, 16
_NW = _NC * _NS
_CHUNK = 128
_T = 256
_SEL_ROWS = 1024


def _sc_gather_wide(w_wide, idx_half, N, D):
    T, C = _T, _CHUNK
    b_per_w = N // _NW
    mesh = plsc.VectorSubcoreMesh(core_axis_name="c", subcore_axis_name="s")

    @pl.kernel(
        out_type=jax.ShapeDtypeStruct((N, 2 * D), jnp.float32),
        mesh=mesh,
        scratch_types=[
            pltpu.VMEM((b_per_w,), jnp.int32),
            pltpu.VMEM((2, T, 2 * D), jnp.float32),
            pltpu.SemaphoreType.DMA,
            pltpu.SemaphoreType.DMA,
            pltpu.SemaphoreType.DMA,
        ],
    )
    def _gather(w_hbm, i_hbm, o_hbm, i_v, s_v, sem_g, sem_o0, sem_o1):
        wid = lax.axis_index("s") * _NC + lax.axis_index("c")
        base = wid * b_per_w
        pltpu.sync_copy(i_hbm.at[pl.ds(base, b_per_w)], i_v)

        def gather_buf(off, buf):
            cps = [
                pltpu.async_copy(
                    w_hbm.at[plsc.Indices(i_v.at[pl.ds(off + k, C)])],
                    buf.at[pl.ds(k, C)], sem_g)
                for k in range(0, T, C)
            ]
            for cp in cps:
                cp.wait()

        def fire_out(off, buf, sem):
            pltpu.async_copy(buf, o_hbm.at[pl.ds(base + off, T)], sem)

        def wait_out(buf, sem):
            pltpu.make_async_copy(buf, o_hbm.at[pl.ds(base, T)], sem).wait()

        gather_buf(0, s_v.at[0])
        fire_out(0, s_v.at[0], sem_o0)
        gather_buf(T, s_v.at[1])
        fire_out(T, s_v.at[1], sem_o1)

        @pl.loop(2 * T, b_per_w, step=2 * T)
        def _(off):
            wait_out(s_v.at[0], sem_o0)
            gather_buf(off, s_v.at[0])
            fire_out(off, s_v.at[0], sem_o0)
            wait_out(s_v.at[1], sem_o1)
            gather_buf(off + T, s_v.at[1])
            fire_out(off + T, s_v.at[1], sem_o1)

        wait_out(s_v.at[0], sem_o0)
        wait_out(s_v.at[1], sem_o1)

    return _gather(w_wide, idx_half)


def _tc_select_half(wide, parity, N, D):
    R = _SEL_ROWS

    def body(w_ref, p_ref, o_ref):
        lo = w_ref[:, :D]
        hi = w_ref[:, D:]
        o_ref[...] = jnp.where(p_ref[...] != 0, hi, lo)

    return pl.pallas_call(
        body,
        grid=(N // R,),
        in_specs=[
            pl.BlockSpec((R, 2 * D), lambda i: (i, 0)),
            pl.BlockSpec((R, 1), lambda i: (i, 0)),
        ],
        out_specs=pl.BlockSpec((R, D), lambda i: (i, 0)),
        out_shape=jax.ShapeDtypeStruct((N, D), jnp.float32),
    )(wide, parity)


def kernel(token_ids, W):
    B, L = token_ids.shape
    V, D = W.shape
    N = B * L
    assert N % (_NW * 2 * _T) == 0 and _T % _CHUNK == 0
    assert V % 2 == 0 and N % _SEL_ROWS == 0

    idx = token_ids.reshape(N)
    idx_half = lax.shift_right_logical(idx, 1)
    parity = lax.bitwise_and(idx, 1).reshape(N, 1)
    w_wide = W.reshape(V // 2, 2 * D)
    wide = _sc_gather_wide(w_wide, idx_half, N, D)
    out = _tc_select_half(wide, parity, N, D)
    return out.reshape(B, L, D)


# speedup vs baseline: 2.1488x; 2.0034x over previous
import jax
import jax.numpy as jnp
from jax import lax
from jax.experimental import pallas as pl
from jax.experimental.pallas import tpu as pltpu
from jax.experimental.pallas import tpu_sc as plsc

_NC, _NS = 2, 16
_NW = _NC * _NS
_CHUNK = 128
_T = 256


def _sc_gather(w128, idx, N, D):
    T, C = _T, _CHUNK
    b_per_w = N // _NW
    mesh = plsc.VectorSubcoreMesh(core_axis_name="c", subcore_axis_name="s")

    @pl.kernel(
        out_type=jax.ShapeDtypeStruct((N, 2 * D), jnp.float32),
        mesh=mesh,
        scratch_types=[
            pltpu.VMEM((b_per_w,), jnp.int32),
            pltpu.VMEM((2, T, 2 * D), jnp.float32),
            pltpu.SemaphoreType.DMA,
            pltpu.SemaphoreType.DMA,
            pltpu.SemaphoreType.DMA,
        ],
    )
    def _gather(w_hbm, i_hbm, o_hbm, i_v, s_v, sem_g, sem_o0, sem_o1):
        wid = lax.axis_index("s") * _NC + lax.axis_index("c")
        base = wid * b_per_w
        pltpu.sync_copy(i_hbm.at[pl.ds(base, b_per_w)], i_v)

        def gather_buf(off, b):
            cps = [
                pltpu.async_copy(
                    w_hbm.at[plsc.Indices(i_v.at[pl.ds(off + k, C)])],
                    s_v.at[b, pl.ds(k, C)], sem_g)
                for k in range(0, T, C)
            ]
            for cp in cps:
                cp.wait()

        def fire_out(off, b, sem):
            pltpu.async_copy(s_v.at[b], o_hbm.at[pl.ds(base + off, T)], sem)

        def wait_out(b, sem):
            pltpu.make_async_copy(s_v.at[b],
                                  o_hbm.at[pl.ds(base, T)], sem).wait()

        gather_buf(0, 0)
        fire_out(0, 0, sem_o0)
        gather_buf(T, 1)
        fire_out(T, 1, sem_o1)

        @pl.loop(2 * T, b_per_w, step=2 * T)
        def _(off):
            wait_out(0, sem_o0)
            gather_buf(off, 0)
            fire_out(off, 0, sem_o0)
            wait_out(1, sem_o1)
            gather_buf(off + T, 1)
            fire_out(off + T, 1, sem_o1)

        wait_out(0, sem_o0)
        wait_out(1, sem_o1)

    return _gather(w128, idx)


def kernel(token_ids, W):
    B, L = token_ids.shape
    V, D = W.shape
    N = B * L
    assert N % (_NW * 2 * _T) == 0 and _T % _CHUNK == 0

    idx = token_ids.reshape(N)
    w128 = jnp.pad(W, ((0, 0), (0, D)))
    out = _sc_gather(w128, idx, N, D)
    return out[:, :D].reshape(B, L, D)


# speedup vs baseline: 2.2786x; 1.0604x over previous
import jax
import jax.numpy as jnp
from jax import lax
from jax.experimental import pallas as pl
from jax.experimental.pallas import tpu as pltpu
from jax.experimental.pallas import tpu_sc as plsc

_NC, _NS = 2, 16
_NW = _NC * _NS
_CHUNK = 128
_T = 256


_CB = 2048


def _tc_pack_table(W, V, D):
    WT = W.T

    def body(wt_ref, o_ref):
        o_ref[:, :D] = jnp.transpose(wt_ref[...])

    return pl.pallas_call(
        body,
        grid=(V // _CB,),
        in_specs=[pl.BlockSpec((D, _CB), lambda i: (0, i))],
        out_specs=pl.BlockSpec((_CB, 2 * D), lambda i: (i, 0)),
        out_shape=jax.ShapeDtypeStruct((V, 2 * D), jnp.float32),
        compiler_params=pltpu.CompilerParams(
            dimension_semantics=("arbitrary",)),
    )(WT)


def _sc_gather(w128, idx, N, D):
    T, C = _T, _CHUNK
    b_per_w = N // _NW
    mesh = plsc.VectorSubcoreMesh(core_axis_name="c", subcore_axis_name="s")

    @pl.kernel(
        out_type=jax.ShapeDtypeStruct((N, 2 * D), jnp.float32),
        mesh=mesh,
        scratch_types=[
            pltpu.VMEM((b_per_w,), jnp.int32),
            pltpu.VMEM((2, T, 2 * D), jnp.float32),
            pltpu.SemaphoreType.DMA,
            pltpu.SemaphoreType.DMA,
            pltpu.SemaphoreType.DMA,
        ],
    )
    def _gather(w_hbm, i_hbm, o_hbm, i_v, s_v, sem_g, sem_o0, sem_o1):
        wid = lax.axis_index("s") * _NC + lax.axis_index("c")
        base = wid * b_per_w
        pltpu.sync_copy(i_hbm.at[pl.ds(base, b_per_w)], i_v)

        def gather_buf(off, b):
            cps = [
                pltpu.async_copy(
                    w_hbm.at[plsc.Indices(i_v.at[pl.ds(off + k, C)])],
                    s_v.at[b, pl.ds(k, C)], sem_g)
                for k in range(0, T, C)
            ]
            for cp in cps:
                cp.wait()

        def fire_out(off, b, sem):
            pltpu.async_copy(s_v.at[b], o_hbm.at[pl.ds(base + off, T)], sem)

        def wait_out(b, sem):
            pltpu.make_async_copy(s_v.at[b],
                                  o_hbm.at[pl.ds(base, T)], sem).wait()

        gather_buf(0, 0)
        fire_out(0, 0, sem_o0)
        gather_buf(T, 1)
        fire_out(T, 1, sem_o1)

        @pl.loop(2 * T, b_per_w, step=2 * T)
        def _(off):
            wait_out(0, sem_o0)
            gather_buf(off, 0)
            fire_out(off, 0, sem_o0)
            wait_out(1, sem_o1)
            gather_buf(off + T, 1)
            fire_out(off + T, 1, sem_o1)

        wait_out(0, sem_o0)
        wait_out(1, sem_o1)

    return _gather(w128, idx)


def kernel(token_ids, W):
    B, L = token_ids.shape
    V, D = W.shape
    N = B * L
    assert N % (_NW * 2 * _T) == 0 and _T % _CHUNK == 0

    idx = token_ids.reshape(N)
    w128 = _tc_pack_table(W, V, D)
    out = _sc_gather(w128, idx, N, D)
    return out[:, :D].reshape(B, L, D)


# speedup vs baseline: 2.2816x; 1.0013x over previous
import jax
import jax.numpy as jnp
from jax import lax
from jax.experimental import pallas as pl
from jax.experimental.pallas import tpu as pltpu
from jax.experimental.pallas import tpu_sc as plsc

_NC, _NS = 2, 16
_NW = _NC * _NS
_CHUNK = 128
_T = 256


_CB = 2048


def _tc_pack_table(W, V, D):
    WT = W.T

    def body(wt_ref, o_ref):
        o_ref[:, :D] = jnp.transpose(wt_ref[...])

    return pl.pallas_call(
        body,
        grid=(pl.cdiv(V, _CB),),
        in_specs=[pl.BlockSpec((D, _CB), lambda i: (0, i))],
        out_specs=pl.BlockSpec((_CB, 2 * D), lambda i: (i, 0)),
        out_shape=jax.ShapeDtypeStruct((V, 2 * D), jnp.float32),
        compiler_params=pltpu.CompilerParams(
            dimension_semantics=("arbitrary",)),
    )(WT)


def _sc_gather(w128, idx, N, D):
    T, C = _T, _CHUNK
    b_per_w = N // _NW
    mesh = plsc.VectorSubcoreMesh(core_axis_name="c", subcore_axis_name="s")

    @pl.kernel(
        out_type=jax.ShapeDtypeStruct((N, 2 * D), jnp.float32),
        mesh=mesh,
        scratch_types=[
            pltpu.VMEM((b_per_w,), jnp.int32),
            pltpu.VMEM((2, T, 2 * D), jnp.float32),
            pltpu.SemaphoreType.DMA,
            pltpu.SemaphoreType.DMA,
            pltpu.SemaphoreType.DMA,
        ],
    )
    def _gather(w_hbm, i_hbm, o_hbm, i_v, s_v, sem_g, sem_o0, sem_o1):
        wid = lax.axis_index("s") * _NC + lax.axis_index("c")
        base = wid * b_per_w
        pltpu.sync_copy(i_hbm.at[pl.ds(base, b_per_w)], i_v)

        def gather_buf(off, b):
            cps = [
                pltpu.async_copy(
                    w_hbm.at[plsc.Indices(i_v.at[pl.ds(off + k, C)])],
                    s_v.at[b, pl.ds(k, C)], sem_g)
                for k in range(0, T, C)
            ]
            for cp in cps:
                cp.wait()

        def fire_out(off, b, sem):
            pltpu.async_copy(s_v.at[b], o_hbm.at[pl.ds(base + off, T)], sem)

        def wait_out(b, sem):
            pltpu.make_async_copy(s_v.at[b],
                                  o_hbm.at[pl.ds(base, T)], sem).wait()

        gather_buf(0, 0)
        fire_out(0, 0, sem_o0)
        gather_buf(T, 1)
        fire_out(T, 1, sem_o1)

        @pl.loop(2 * T, b_per_w, step=2 * T)
        def _(off):
            wait_out(0, sem_o0)
            gather_buf(off, 0)
            fire_out(off, 0, sem_o0)
            wait_out(1, sem_o1)
            gather_buf(off + T, 1)
            fire_out(off + T, 1, sem_o1)

        wait_out(0, sem_o0)
        wait_out(1, sem_o1)

    return _gather(w128, idx)


def kernel(token_ids, W):
    B, L = token_ids.shape
    V, D = W.shape
    N = B * L
    assert N % (_NW * 2 * _T) == 0 and _T % _CHUNK == 0

    idx = token_ids.reshape(N)
    w128 = _tc_pack_table(W, V, D)
    out = _sc_gather(w128, idx, N, D)
    return out[:, :D].reshape(B, L, D)
